# baseline (speedup 1.0000x reference)
import functools

import jax
import jax.numpy as jnp
from jax import lax
from jax.experimental import pallas as pl
from jax.experimental.pallas import tpu as pltpu
from jax.experimental.pallas import tpu_sc as plsc


def _transpose_table(tt, V, D):
    BL = 512

    def body(in_ref, out_ref):
        out_ref[...] = in_ref[...].T

    return pl.pallas_call(
        body,
        grid=(pl.cdiv(V, BL),),
        in_specs=[pl.BlockSpec((D, BL), lambda i: (0, i))],
        out_specs=pl.BlockSpec((BL, D), lambda i: (i, 0)),
        out_shape=jax.ShapeDtypeStruct((V, D), jnp.float32),
    )(tt)


def _relayout_out(rows, L, NV, B, D):
    BB = 512

    def body(in_ref, out_ref):
        out_ref[0] = in_ref[...].T

    return pl.pallas_call(
        body,
        grid=(L, NV, B // BB),
        in_specs=[
            pl.BlockSpec(
                (BB, D), lambda l, v, b: ((l * NV + v) * (B // BB) + b, 0)
            )
        ],
        out_specs=pl.BlockSpec((1, D, BB), lambda l, v, b: (l, v, b)),
        out_shape=jax.ShapeDtypeStruct((L, NV * D, B), jnp.float32),
    )(rows)


def _build_gather(N, V, D, nc, ns):
    NW = nc * ns
    n_per_w = N // NW
    C = 800
    NBUF = 4
    K = 2
    assert n_per_w % C == 0
    nchunks = n_per_w // C

    mesh = plsc.VectorSubcoreMesh(core_axis_name="c", subcore_axis_name="s")

    @functools.partial(
        pl.kernel,
        out_type=jax.ShapeDtypeStruct((N, D), jnp.float32),
        mesh=mesh,
        compiler_params=pltpu.CompilerParams(use_tc_tiling_on_sc=False),
        scratch_types=[
            pltpu.VMEM((n_per_w,), jnp.int32),
            pltpu.VMEM((NBUF, C, D), jnp.float32),
            [pltpu.SemaphoreType.DMA] * NBUF,
            [pltpu.SemaphoreType.DMA] * NBUF,
        ],
    )
    def gather_kernel(idx_hbm, table_hbm, out_hbm, idx_v, rows_v, gsem, wsem):
        wid = lax.axis_index("s") * nc + lax.axis_index("c")
        base = wid * n_per_w
        pltpu.sync_copy(idx_hbm.at[pl.ds(base, n_per_w)], idx_v)

        def gather(i):
            b = i % NBUF
            return pltpu.async_copy(
                table_hbm.at[idx_v.at[pl.ds(i * C, C)]], rows_v.at[b], gsem[b]
            )

        def writeback(i):
            b = i % NBUF
            return pltpu.async_copy(
                rows_v.at[b], out_hbm.at[pl.ds(base + i * C, C)], wsem[b]
            )

        gh = [None] * nchunks
        wh = [None] * nchunks
        for i in range(nchunks + K):
            if i < nchunks:
                if i >= NBUF:
                    wh[i - NBUF].wait()
                gh[i] = gather(i)
            j = i - K
            if 0 <= j:
                gh[j].wait()
                wh[j] = writeback(j)
        for j in range(nchunks - NBUF, nchunks):
            wh[j].wait()

    return gather_kernel


def kernel(indices, mask, table):
    B, L, NV = indices.shape
    V, D = table.shape
    N = B * L * NV
    info = plsc.get_sparse_core_info()
    gather_fn = _build_gather(N, V, D, info.num_cores, info.num_subcores)
    tl = _transpose_table(table.T, V, D)
    idx_flat = jnp.transpose(indices, (1, 2, 0)).reshape(N)
    rows = gather_fn(idx_flat, tl)
    outp = _relayout_out(rows, L, NV, B, D)
    return jnp.transpose(outp, (2, 0, 1)), mask


# speedup vs baseline: 2.4374x; 2.4374x over previous
import functools

import jax
import jax.numpy as jnp
from jax import lax
from jax.experimental import pallas as pl
from jax.experimental.pallas import tpu as pltpu
from jax.experimental.pallas import tpu_sc as plsc


def _transpose_table(tt, V, D):
    BL = 8192

    def body(in_ref, out_ref):
        out_ref[...] = in_ref[...].T

    return pl.pallas_call(
        body,
        grid=(pl.cdiv(V, BL),),
        in_specs=[pl.BlockSpec((D, BL), lambda i: (0, i))],
        out_specs=pl.BlockSpec((BL, D), lambda i: (i, 0)),
        out_shape=jax.ShapeDtypeStruct((V, D), jnp.float32),
    )(tt)


def _relayout_out(rows, L, NV, B, D):
    def body(in_ref, out_ref):
        out_ref[0] = in_ref[...].T

    return pl.pallas_call(
        body,
        grid=(L, NV),
        in_specs=[pl.BlockSpec((B, D), lambda l, v: (l * NV + v, 0))],
        out_specs=pl.BlockSpec((1, D, B), lambda l, v: (l, v, 0)),
        out_shape=jax.ShapeDtypeStruct((L, NV * D, B), jnp.float32),
    )(rows)


def _build_gather(N, V, D, nc, ns):
    NW = nc * ns
    n_per_w = N // NW
    C = 800
    NBUF = 4
    K = 2
    assert n_per_w % C == 0
    nchunks = n_per_w // C

    mesh = plsc.VectorSubcoreMesh(core_axis_name="c", subcore_axis_name="s")

    @functools.partial(
        pl.kernel,
        out_type=jax.ShapeDtypeStruct((N, D), jnp.float32),
        mesh=mesh,
        compiler_params=pltpu.CompilerParams(use_tc_tiling_on_sc=False),
        scratch_types=[
            pltpu.VMEM((n_per_w,), jnp.int32),
            pltpu.VMEM((NBUF, C, D), jnp.float32),
            [pltpu.SemaphoreType.DMA] * NBUF,
            [pltpu.SemaphoreType.DMA] * NBUF,
        ],
    )
    def gather_kernel(idx_hbm, table_hbm, out_hbm, idx_v, rows_v, gsem, wsem):
        wid = lax.axis_index("s") * nc + lax.axis_index("c")
        base = wid * n_per_w
        pltpu.sync_copy(idx_hbm.at[pl.ds(base, n_per_w)], idx_v)

        def gather(i):
            b = i % NBUF
            return pltpu.async_copy(
                table_hbm.at[idx_v.at[pl.ds(i * C, C)]], rows_v.at[b], gsem[b]
            )

        def writeback(i):
            b = i % NBUF
            return pltpu.async_copy(
                rows_v.at[b], out_hbm.at[pl.ds(base + i * C, C)], wsem[b]
            )

        gh = [None] * nchunks
        wh = [None] * nchunks
        for i in range(nchunks + K):
            if i < nchunks:
                if i >= NBUF:
                    wh[i - NBUF].wait()
                gh[i] = gather(i)
            j = i - K
            if 0 <= j:
                gh[j].wait()
                wh[j] = writeback(j)
        for j in range(nchunks - NBUF, nchunks):
            wh[j].wait()

    return gather_kernel


def kernel(indices, mask, table):
    B, L, NV = indices.shape
    V, D = table.shape
    N = B * L * NV
    info = plsc.get_sparse_core_info()
    gather_fn = _build_gather(N, V, D, info.num_cores, info.num_subcores)
    tl = _transpose_table(table.T, V, D)
    idx_flat = jnp.transpose(indices, (1, 2, 0)).reshape(N)
    rows = gather_fn(idx_flat, tl)
    outp = _relayout_out(rows, L, NV, B, D)
    return jnp.transpose(outp, (2, 0, 1)), mask
